# baseline (speedup 1.0000x reference)
import jax
import jax.numpy as jnp
from jax import lax
from jax.experimental import pallas as pl
from jax.experimental.pallas import tpu as pltpu
from jax.experimental.pallas import tpu_sc as plsc

VOCAB = 1000000
EMBED_DIM = 64
NUM_CLASSES = 2
BATCH = 4096
HIST = 200

NC = 2
NS = 16
NW = NC * NS
ROWS_PER_W = BATCH // NW
CH = (104, 96)
CHOFF = (0, 104)
CHMAX = 104
CHUNKS_PER_ROW = 2
NCHUNKS = ROWS_PER_W * CHUNKS_PER_ROW
D16 = EMBED_DIM // 16
NBUF = 4


def _sc_body(table_hbm, x_hbm, wt_hbm, b_hbm, out_hbm,
             idx_v, bufs, wt_v, b_v, out_v, sems):
    cid = lax.axis_index("c")
    sid = lax.axis_index("s")
    wid = sid * NC + cid

    pltpu.sync_copy(x_hbm.at[pl.ds(wid * ROWS_PER_W, ROWS_PER_W)], idx_v)
    pltpu.sync_copy(wt_hbm, wt_v)
    pltpu.sync_copy(b_hbm, b_v)

    lane = lax.broadcasted_iota(jnp.int32, (16,), 0)
    zero = jnp.zeros((16,), jnp.float32)
    b_vec = b_v[...]
    wvecs = tuple(wt_v[c, pl.ds(k * 16, 16)]
                  for c in range(NUM_CLASSES) for k in range(D16))
    inv_l = jnp.float32(1.0 / HIST)

    def fire(slot, row, half):
        n = CH[half]
        idx = idx_v.at[row, pl.ds(CHOFF[half], n)]
        return pltpu.async_copy(table_hbm.at[idx],
                                bufs.at[slot, pl.ds(0, n)],
                                sems.at[slot])

    def wait(slot, half):
        n = CH[half]
        pltpu.make_async_copy(table_hbm.at[idx_v.at[0, pl.ds(0, n)]],
                              bufs.at[slot, pl.ds(0, n)], sems.at[slot]).wait()

    def reduce_buf(slot, half, acc):
        buf = bufs.at[slot]

        @plsc.parallel_loop(0, CH[half], step=1, unroll=4, carry=acc)
        def body(r, a):
            return tuple(a[k] + buf[r, pl.ds(k * 16, 16)] for k in range(D16))

        return body

    def finalize(row, acc):
        out_row = b_vec
        for c in range(NUM_CLASSES):
            s = jnp.float32(0.0)
            for k in range(D16):
                s = s + jnp.sum(acc[k] * wvecs[c * D16 + k])
            out_row = out_row + jnp.where(lane == c, s * inv_l, 0.0)
        out_v[row] = out_row

    for b in range(NBUF):
        fire(b, b // 2, b % 2)

    @pl.loop(0, NCHUNKS - NBUF, step=NBUF)
    def _(g):
        row = g >> 1
        for b in range(NBUF):
            wait(b, b % 2)
            acc = (zero,) * D16 if b % 2 == 0 else acc2
            acc2 = reduce_buf(b, b % 2, acc)
            nxt = g + b + NBUF
            fire(b, nxt >> 1, b % 2)
            if b % 2 == 1:
                finalize(row + b // 2, acc2)

    for b in range(NBUF):
        j = NCHUNKS - NBUF + b
        wait(b, b % 2)
        acc = (zero,) * D16 if b % 2 == 0 else acc2
        acc2 = reduce_buf(b, b % 2, acc)
        if b % 2 == 1:
            finalize(j // 2, acc2)

    pltpu.sync_copy(out_v, out_hbm.at[pl.ds(wid * ROWS_PER_W, ROWS_PER_W)])


_sc_call = pl.kernel(
    _sc_body,
    out_type=jax.ShapeDtypeStruct((BATCH, 16), jnp.float32),
    mesh=plsc.VectorSubcoreMesh(core_axis_name="c", subcore_axis_name="s"),
    scratch_types=[
        pltpu.VMEM((ROWS_PER_W, HIST), jnp.int32),
        pltpu.VMEM((NBUF, CHMAX, EMBED_DIM), jnp.float32),
        pltpu.VMEM((NUM_CLASSES, EMBED_DIM), jnp.float32),
        pltpu.VMEM((16,), jnp.float32),
        pltpu.VMEM((ROWS_PER_W, 16), jnp.float32),
        pltpu.SemaphoreType.DMA((NBUF,)),
    ],
    compiler_params=pltpu.CompilerParams(
        needs_layout_passes=False, use_tc_tiling_on_sc=False),
)


@jax.jit
def kernel(x, table, W, b):
    wt = W.T.astype(jnp.float32)
    b_pad = jnp.pad(b.astype(jnp.float32), (0, 16 - NUM_CLASSES))
    out16 = _sc_call(table, x.astype(jnp.int32), wt, b_pad)
    return out16[:, :NUM_CLASSES]
